# baseline (speedup 1.0000x reference)
import functools
import math

import jax
import jax.numpy as jnp
from jax import lax
from jax.experimental import pallas as pl
from jax.experimental.pallas import tpu as pltpu
from jax.experimental.pallas import tpu_sc as plsc

EMBED_DIM = 64
SCALE = math.sqrt(EMBED_DIM)

NUM_CORES = 2
NUM_SUBCORES = 16
NUM_WORKERS = NUM_CORES * NUM_SUBCORES

IDX_GRAIN = 128
CHUNK = 512
K = CHUNK // IDX_GRAIN
LANES = 16


def _build(total_rows: int):
    rows_per_worker = total_rows // NUM_WORKERS
    chunks_per_worker = rows_per_worker // CHUNK
    idx_rows_per_worker = rows_per_worker // IDX_GRAIN

    mesh = plsc.VectorSubcoreMesh(core_axis_name="c", subcore_axis_name="s")

    @functools.partial(
        pl.kernel,
        out_type=jax.ShapeDtypeStruct((total_rows, EMBED_DIM), jnp.float32),
        mesh=mesh,
        scratch_types=[
            pltpu.VMEM((K, IDX_GRAIN), jnp.int32),
            pltpu.VMEM((CHUNK, EMBED_DIM), jnp.float32),
            pltpu.SemaphoreType.DMA,
        ],
        compiler_params=pltpu.CompilerParams(use_tc_tiling_on_sc=False),
    )
    def emb(tokens_hbm, table_hbm, out_hbm, idx_v, rows_v, sem):
        wid = lax.axis_index("s") * NUM_CORES + lax.axis_index("c")
        base_row = wid * rows_per_worker
        base_idx_row = wid * idx_rows_per_worker

        def chunk_body(g, carry):
            pltpu.sync_copy(tokens_hbm.at[pl.ds(base_idx_row + g * K, K)],
                            idx_v)
            copies = [
                pltpu.async_copy(
                    table_hbm.at[idx_v.at[j]],
                    rows_v.at[pl.ds(j * IDX_GRAIN, IDX_GRAIN)],
                    sem,
                )
                for j in range(K)
            ]
            for c in copies:
                c.wait()

            def scale_body(i, carry2):
                for j in range(EMBED_DIM // LANES):
                    sl = (i, pl.ds(j * LANES, LANES))
                    rows_v[sl] = rows_v[sl] * SCALE
                return carry2

            lax.fori_loop(0, CHUNK, scale_body, 0, unroll=4)

            pltpu.sync_copy(rows_v,
                            out_hbm.at[pl.ds(base_row + g * CHUNK, CHUNK)])
            return carry

        lax.fori_loop(0, chunks_per_worker, chunk_body, 0)

    return emb


def kernel(tokens, table):
    b, s = tokens.shape
    total_rows = b * s
    tokens2d = tokens.reshape(total_rows // IDX_GRAIN, IDX_GRAIN)
    out = _build(total_rows)(tokens2d, table)
    return out.reshape(b, s, EMBED_DIM)


# speedup vs baseline: 1.0830x; 1.0830x over previous
import functools
import math

import jax
import jax.numpy as jnp
from jax import lax
from jax.experimental import pallas as pl
from jax.experimental.pallas import tpu as pltpu
from jax.experimental.pallas import tpu_sc as plsc

EMBED_DIM = 64
SCALE = math.sqrt(EMBED_DIM)

NUM_CORES = 2
NUM_SUBCORES = 16
NUM_WORKERS = NUM_CORES * NUM_SUBCORES

IDX_GRAIN = 128
CHUNK = 256
K = CHUNK // IDX_GRAIN
NBUF = 5
LANES = 16


def _build(total_rows: int):
    rows_per_worker = total_rows // NUM_WORKERS
    idx_rows_per_worker = rows_per_worker // IDX_GRAIN
    chunks_per_worker = rows_per_worker // CHUNK
    rounds = chunks_per_worker // NBUF
    assert rounds * NBUF == chunks_per_worker

    mesh = plsc.VectorSubcoreMesh(core_axis_name="c", subcore_axis_name="s")

    @functools.partial(
        pl.kernel,
        out_type=jax.ShapeDtypeStruct((total_rows, EMBED_DIM), jnp.float32),
        mesh=mesh,
        scratch_types=[
            pltpu.VMEM((idx_rows_per_worker, IDX_GRAIN), jnp.int32),
            pltpu.VMEM((NBUF, CHUNK, EMBED_DIM), jnp.float32),
        ] + [pltpu.SemaphoreType.DMA] * (2 * NBUF),
        compiler_params=pltpu.CompilerParams(use_tc_tiling_on_sc=False),
    )
    def emb(tokens_hbm, table_hbm, out_hbm, idx_all, rows, *sems):
        sem_in = sems[:NBUF]
        sem_out = sems[NBUF:]
        wid = lax.axis_index("s") * NUM_CORES + lax.axis_index("c")
        base_row = wid * rows_per_worker

        pltpu.sync_copy(
            tokens_hbm.at[pl.ds(wid * idx_rows_per_worker,
                                idx_rows_per_worker)],
            idx_all)

        def round_body(p, carry):
            c0 = p * NBUF
            gathers = []
            for b in range(NBUF):
                g = c0 + b
                for j in range(K):
                    gathers.append(pltpu.async_copy(
                        table_hbm.at[idx_all.at[g * K + j]],
                        rows.at[b, pl.ds(j * IDX_GRAIN, IDX_GRAIN)],
                        sem_in[b]))
            writebacks = []
            for b in range(NBUF):
                g = c0 + b
                for j in range(K):
                    gathers[b * K + j].wait()

                def scale_body(i, carry2):
                    for j in range(EMBED_DIM // LANES):
                        sl = (b, i, pl.ds(j * LANES, LANES))
                        rows[sl] = rows[sl] * SCALE
                    return carry2

                lax.fori_loop(0, CHUNK, scale_body, 0, unroll=4)
                writebacks.append(pltpu.async_copy(
                    rows.at[b],
                    out_hbm.at[pl.ds(base_row + g * CHUNK, CHUNK)],
                    sem_out[b]))
            for wb in writebacks:
                wb.wait()
            return carry

        lax.fori_loop(0, rounds, round_body, 0)

    return emb


def kernel(tokens, table):
    b, s = tokens.shape
    total_rows = b * s
    tokens2d = tokens.reshape(total_rows // IDX_GRAIN, IDX_GRAIN)
    out = _build(total_rows)(tokens2d, table)
    return out.reshape(b, s, EMBED_DIM)
